# baseline (speedup 1.0000x reference)
import functools

import jax
import jax.numpy as jnp
from jax import lax
from jax.experimental import pallas as pl
from jax.experimental.pallas import tpu as pltpu

N_JRNL = 1000000
JRNL_DIM = 32
MESH_SIZE = 28340
HIDDEN_SIZE = 128
N_PROBES = 3
BATCH = 1024
CTX_DIM = HIDDEN_SIZE * N_PROBES
IN_FEAT = CTX_DIM + JRNL_DIM

_BLK_N = 2048

_GRP = 16
_NGRP = BATCH // _GRP
_LANES = 128


def _gather_body(c_ref, tbl_ref, l_ref, out_ref, g_ref, sem):
    def _issue(j, _):
        c = c_ref[j, 0] // _LANES
        pltpu.make_async_copy(
            tbl_ref.at[:, pl.ds(c * _LANES, _LANES)], g_ref.at[j], sem,
        ).start()
        return 0

    lax.fori_loop(0, BATCH, _issue, 0, unroll=8)

    for gq in range(_NGRP):
        grp = g_ref.at[pl.ds(_GRP * gq, _GRP)]
        pltpu.make_async_copy(grp, grp, sem).wait()

    lane = lax.broadcasted_iota(jnp.int32, (_GRP, _GRP * _LANES), 1)
    row = lax.broadcasted_iota(jnp.int32, (_GRP, _GRP * _LANES), 0)
    for gq in range(_NGRP):
        chunks = jnp.concatenate(
            [g_ref[_GRP * gq + k] for k in range(_GRP)], axis=1)
        sel = ((lane % _LANES) == (l_ref[gq] % _LANES)) & ((lane // _LANES) == row)
        cols = lax.dot_general(
            chunks, sel.astype(jnp.float32), (((1,), (1,)), ((), ())),
            preferred_element_type=jnp.float32,
        )
        out_ref[:, _GRP * gq:_GRP * (gq + 1)] = cols


def _mm_body(wt_ref, ctx_ref, emb_ref, b_ref, out_ref, comb_ref):
    @pl.when(pl.program_id(0) == 0)
    def _init():
        comb_ref[0:CTX_DIM, :] = jnp.transpose(ctx_ref[...])
        comb_ref[CTX_DIM:IN_FEAT, :] = emb_ref[...]

    acc = lax.dot_general(
        wt_ref[...], comb_ref[...],
        (((0,), (0,)), ((), ())),
        preferred_element_type=jnp.float32,
    )
    out_ref[...] = acc + jnp.transpose(b_ref[...])


@jax.jit
def kernel(jrnl_variable, context_vectors, emb_table, W, b):
    idx3 = jrnl_variable.reshape((_NGRP, _GRP, 1))
    table_t = emb_table.T
    wt = W.T
    b2d = b.reshape((1, MESH_SIZE))

    emb_t = pl.pallas_call(
        _gather_body,
        grid_spec=pltpu.PrefetchScalarGridSpec(
            num_scalar_prefetch=1,
            grid=(1,),
            in_specs=[
                pl.BlockSpec(memory_space=pl.ANY),
                pl.BlockSpec((_NGRP, _GRP, 1), lambda g, c_ref: (0, 0, 0)),
            ],
            out_specs=pl.BlockSpec((JRNL_DIM, BATCH), lambda g, c_ref: (0, 0)),
            scratch_shapes=[
                pltpu.VMEM((BATCH, JRNL_DIM, _LANES), jnp.float32),
                pltpu.SemaphoreType.DMA,
            ],
        ),
        out_shape=jax.ShapeDtypeStruct((JRNL_DIM, BATCH), jnp.float32),
        compiler_params=pltpu.CompilerParams(
            dimension_semantics=("arbitrary",),
        ),
    )(jrnl_variable, table_t, idx3)

    n_blocks = pl.cdiv(MESH_SIZE, _BLK_N)
    out_t = pl.pallas_call(
        _mm_body,
        grid=(n_blocks,),
        in_specs=[
            pl.BlockSpec((IN_FEAT, _BLK_N), lambda i: (0, i)),
            pl.BlockSpec((BATCH, CTX_DIM), lambda i: (0, 0)),
            pl.BlockSpec((JRNL_DIM, BATCH), lambda i: (0, 0)),
            pl.BlockSpec((1, _BLK_N), lambda i: (0, i)),
        ],
        out_specs=pl.BlockSpec((_BLK_N, BATCH), lambda i: (i, 0)),
        out_shape=jax.ShapeDtypeStruct((MESH_SIZE, BATCH), jnp.float32),
        scratch_shapes=[pltpu.VMEM((IN_FEAT, BATCH), jnp.float32)],
        compiler_params=pltpu.CompilerParams(
            dimension_semantics=("arbitrary",),
        ),
    )(wt, context_vectors, emb_t, b2d)
    return out_t.T
